# baseline (speedup 1.0000x reference)
import jax
import jax.numpy as jnp
from jax.experimental import pallas as pl
from jax.experimental.pallas import tpu as pltpu

DIM_CODES = 64
DICT_SIZE = 8192
DIM_EMBED = 32
BATCH = 128
K_CHUNK = 512


def _argmin_body(xt_ref, d_ref, idx_ref, ce_ref):
    xt = xt_ref[0]
    dc = d_ref[0]
    xyT = jax.lax.dot_general(dc, xt, (((1,), (0,)), ((), ())),
                              preferred_element_type=jnp.float32)
    y_sq = jnp.sum(dc * dc, axis=1, keepdims=True)
    x_sq = jnp.sum(xt * xt, axis=0, keepdims=True)
    distT = x_sq - 2.0 * xyT + y_sq
    m = jnp.min(distT, axis=0, keepdims=True)
    kio = jax.lax.broadcasted_iota(jnp.int32, (DICT_SIZE, BATCH), 0)
    cand = jnp.where(distT == m, kio, DICT_SIZE)
    idxv = jnp.min(cand, axis=0, keepdims=True)
    idx_ref[0] = idxv
    onehotT = (kio == idxv).astype(jnp.float32)
    ceT = jax.lax.dot_general(dc, onehotT, (((0,), (0,)), ((), ())),
                              preferred_element_type=jnp.float32,
                              precision=jax.lax.Precision.HIGHEST)
    ce_ref[0] = ceT


def _onehot_body(idx_ref, out_ref):
    k0 = pl.program_id(0) * K_CHUNK
    kio = jax.lax.broadcasted_iota(jnp.int32, (BATCH, DIM_CODES, K_CHUNK), 2) + k0
    out_ref[...] = (kio == idx_ref[...][:, :, None]).astype(jnp.float32)


def kernel(x, dictionary):
    xt = x.reshape(BATCH, DIM_CODES, DIM_EMBED).transpose(1, 2, 0)

    idx_t, ce_t = pl.pallas_call(
        _argmin_body,
        grid=(DIM_CODES,),
        in_specs=[
            pl.BlockSpec((1, DIM_EMBED, BATCH), lambda c: (c, 0, 0)),
            pl.BlockSpec((1, DICT_SIZE, DIM_EMBED), lambda c: (c, 0, 0)),
        ],
        out_specs=[
            pl.BlockSpec((1, 1, BATCH), lambda c: (c, 0, 0)),
            pl.BlockSpec((1, DIM_EMBED, BATCH), lambda c: (c, 0, 0)),
        ],
        out_shape=[
            jax.ShapeDtypeStruct((DIM_CODES, 1, BATCH), jnp.int32),
            jax.ShapeDtypeStruct((DIM_CODES, DIM_EMBED, BATCH), jnp.float32),
        ],
    )(xt, dictionary)

    idx = idx_t.reshape(DIM_CODES, BATCH).transpose(1, 0)
    cw_e = ce_t.transpose(2, 0, 1).reshape(BATCH, DIM_CODES * DIM_EMBED)

    one_hot = pl.pallas_call(
        _onehot_body,
        grid=(DICT_SIZE // K_CHUNK,),
        in_specs=[pl.BlockSpec((BATCH, DIM_CODES), lambda k: (0, 0))],
        out_specs=pl.BlockSpec((BATCH, DIM_CODES, K_CHUNK), lambda k: (0, 0, k)),
        out_shape=jax.ShapeDtypeStruct((BATCH, DIM_CODES, DICT_SIZE), jnp.float32),
    )(idx)

    return cw_e, cw_e, one_hot


# speedup vs baseline: 1.2826x; 1.2826x over previous
import functools

import jax
import jax.numpy as jnp
from jax import lax
from jax.experimental import pallas as pl
from jax.experimental.pallas import tpu as pltpu
from jax.experimental.pallas import tpu_sc as plsc

DIM_CODES = 64
DICT_SIZE = 8192
DIM_EMBED = 32
BATCH = 128
K_CHUNK = 512

_NC = 2
_NW = 32
_ROWS = BATCH * DIM_CODES
_ROWS_PER_W = _ROWS // _NW


def _argmin_body(xt_ref, d_ref, idx_ref, ce_ref):
    xt = xt_ref[0]
    dc = d_ref[0]
    xyT = jax.lax.dot_general(dc, xt, (((1,), (0,)), ((), ())),
                              preferred_element_type=jnp.float32)
    y_sq = jnp.sum(dc * dc, axis=1, keepdims=True)
    x_sq = jnp.sum(xt * xt, axis=0, keepdims=True)
    distT = x_sq - 2.0 * xyT + y_sq
    m = jnp.min(distT, axis=0, keepdims=True)
    kio = jax.lax.broadcasted_iota(jnp.int32, (DICT_SIZE, BATCH), 0)
    cand = jnp.where(distT == m, kio, DICT_SIZE)
    idxv = jnp.min(cand, axis=0, keepdims=True)
    idx_ref[0] = idxv
    onehotT = (kio == idxv).astype(jnp.float32)
    ceT = jax.lax.dot_general(dc, onehotT, (((0,), (0,)), ((), ())),
                              preferred_element_type=jnp.float32)
    ce_ref[0] = ceT


def _onehot_body(idx_ref, out_ref):
    k0 = pl.program_id(0) * K_CHUNK
    kio = jax.lax.broadcasted_iota(jnp.int32, (BATCH, DIM_CODES, K_CHUNK), 2) + k0
    out_ref[...] = (kio == idx_ref[...][:, :, None]).astype(jnp.float32)


def _make_sc_gather():
    @functools.partial(
        pl.kernel,
        mesh=plsc.VectorSubcoreMesh(core_axis_name="c", subcore_axis_name="s"),
        out_type=jax.ShapeDtypeStruct((_ROWS, DIM_EMBED), jnp.float32),
        scratch_types=[
            pltpu.VMEM((_ROWS_PER_W,), jnp.int32),
            pltpu.VMEM((_ROWS_PER_W, DIM_EMBED), jnp.float32),
            pltpu.SemaphoreType.DMA,
        ],
    )
    def _sc_gather(table_hbm, idx_hbm, out_hbm, idx_v, rows_v, sem):
        wid = lax.axis_index("s") * _NC + lax.axis_index("c")
        base = wid * _ROWS_PER_W
        pltpu.sync_copy(idx_hbm.at[pl.ds(base, _ROWS_PER_W)], idx_v)
        pltpu.async_copy(table_hbm.at[idx_v], rows_v, sem).wait()
        pltpu.sync_copy(rows_v, out_hbm.at[pl.ds(base, _ROWS_PER_W)])

    return _sc_gather


def kernel(x, dictionary):
    xt = x.reshape(BATCH, DIM_CODES, DIM_EMBED).transpose(1, 2, 0)

    idx_t, ce_t = pl.pallas_call(
        _argmin_body,
        grid=(DIM_CODES,),
        in_specs=[
            pl.BlockSpec((1, DIM_EMBED, BATCH), lambda c: (c, 0, 0)),
            pl.BlockSpec((1, DICT_SIZE, DIM_EMBED), lambda c: (c, 0, 0)),
        ],
        out_specs=[
            pl.BlockSpec((1, 1, BATCH), lambda c: (c, 0, 0)),
            pl.BlockSpec((1, DIM_EMBED, BATCH), lambda c: (c, 0, 0)),
        ],
        out_shape=[
            jax.ShapeDtypeStruct((DIM_CODES, 1, BATCH), jnp.int32),
            jax.ShapeDtypeStruct((DIM_CODES, DIM_EMBED, BATCH), jnp.float32),
        ],
    )(xt, dictionary)

    idx = idx_t.reshape(DIM_CODES, BATCH).transpose(1, 0)
    cw_e = ce_t.transpose(2, 0, 1).reshape(BATCH, DIM_CODES * DIM_EMBED)

    one_hot = pl.pallas_call(
        _onehot_body,
        grid=(DICT_SIZE // K_CHUNK,),
        in_specs=[pl.BlockSpec((BATCH, DIM_CODES), lambda k: (0, 0))],
        out_specs=pl.BlockSpec((BATCH, DIM_CODES, K_CHUNK), lambda k: (0, 0, k)),
        out_shape=jax.ShapeDtypeStruct((BATCH, DIM_CODES, DICT_SIZE), jnp.float32),
    )(idx)

    return cw_e, cw_e, one_hot


# speedup vs baseline: 1.3884x; 1.0824x over previous
import jax
import jax.numpy as jnp
from jax.experimental import pallas as pl
from jax.experimental.pallas import tpu as pltpu

DIM_CODES = 64
DICT_SIZE = 8192
DIM_EMBED = 32
BATCH = 128


def _fused_body(xt_ref, d_ref, idx_ref, ce_ref, oh_hbm, oh_ref, sem):
    c = pl.program_id(0)
    slot = jax.lax.rem(c, 2)

    xt = xt_ref[0]
    dc = d_ref[0]
    xyT = jax.lax.dot_general(dc, xt, (((1,), (0,)), ((), ())),
                              preferred_element_type=jnp.float32)
    y_sq = jnp.sum(dc * dc, axis=1, keepdims=True)
    x_sq = jnp.sum(xt * xt, axis=0, keepdims=True)
    distT = x_sq - 2.0 * xyT + y_sq
    m = jnp.min(distT, axis=0, keepdims=True)
    kio = jax.lax.broadcasted_iota(jnp.int32, (DICT_SIZE, BATCH), 0)
    cand = jnp.where(distT == m, kio, DICT_SIZE)
    idxv = jnp.min(cand, axis=0, keepdims=True)
    idx_ref[0] = idxv
    onehotT = (kio == idxv).astype(jnp.float32)
    ceT = jax.lax.dot_general(dc, onehotT, (((0,), (0,)), ((), ())),
                              preferred_element_type=jnp.float32)
    ce_ref[0] = ceT

    idx_col = jnp.transpose(idxv)
    kio2 = jax.lax.broadcasted_iota(jnp.int32, (BATCH, DICT_SIZE), 1)

    @pl.when(c >= 2)
    def _wait_prev():
        pltpu.make_async_copy(oh_ref.at[slot], oh_hbm.at[:, c - 2, :],
                              sem.at[slot]).wait()

    oh_ref[slot] = (kio2 == idx_col).astype(jnp.float32)
    pltpu.make_async_copy(oh_ref.at[slot], oh_hbm.at[:, c, :],
                          sem.at[slot]).start()

    @pl.when(c == DIM_CODES - 1)
    def _drain():
        pltpu.make_async_copy(oh_ref.at[0], oh_hbm.at[:, DIM_CODES - 2, :],
                              sem.at[0]).wait()
        pltpu.make_async_copy(oh_ref.at[1], oh_hbm.at[:, DIM_CODES - 1, :],
                              sem.at[1]).wait()


def kernel(x, dictionary):
    xt = x.reshape(BATCH, DIM_CODES, DIM_EMBED).transpose(1, 2, 0)

    idx_t, ce_t, one_hot = pl.pallas_call(
        _fused_body,
        grid=(DIM_CODES,),
        in_specs=[
            pl.BlockSpec((1, DIM_EMBED, BATCH), lambda c: (c, 0, 0)),
            pl.BlockSpec((1, DICT_SIZE, DIM_EMBED), lambda c: (c, 0, 0)),
        ],
        out_specs=[
            pl.BlockSpec((1, 1, BATCH), lambda c: (c, 0, 0)),
            pl.BlockSpec((1, DIM_EMBED, BATCH), lambda c: (c, 0, 0)),
            pl.BlockSpec(memory_space=pltpu.MemorySpace.HBM),
        ],
        out_shape=[
            jax.ShapeDtypeStruct((DIM_CODES, 1, BATCH), jnp.int32),
            jax.ShapeDtypeStruct((DIM_CODES, DIM_EMBED, BATCH), jnp.float32),
            jax.ShapeDtypeStruct((BATCH, DIM_CODES, DICT_SIZE), jnp.float32),
        ],
        scratch_shapes=[
            pltpu.VMEM((2, BATCH, DICT_SIZE), jnp.float32),
            pltpu.SemaphoreType.DMA((2,)),
        ],
    )(xt, dictionary)

    cw_e = ce_t.transpose(2, 0, 1).reshape(BATCH, DIM_CODES * DIM_EMBED)
    return cw_e, cw_e, one_hot
